# baseline (speedup 1.0000x reference)
import functools

import jax
import jax.numpy as jnp
import numpy as np
from jax import lax
from jax.experimental import pallas as pl
from jax.experimental.pallas import tpu as pltpu
from jax.experimental.pallas import tpu_sc as plsc

B = 16384
L = 50
V = 1000000
LP = 56
D = 64
DP = 2 * D
NC, NS = 2, 16
NW = NC * NS
RPW = B // NW
INV_L = 1.0 / L

NBUF = 8
HALF = RPW // 2
NGRP = HALF // NBUF

_Q, _J = np.divmod(np.arange(D), 16)
POS_A = 32 * _Q + _J
POS_B = POS_A + 16


CB = 4096


def _fuse_body(a_ref, b_ref, o_ref):
  a16 = lax.bitcast_convert_type(
      a_ref[...].T.astype(jnp.bfloat16), jnp.uint16).astype(jnp.uint32)
  b16 = lax.bitcast_convert_type(
      b_ref[...].T.astype(jnp.bfloat16), jnp.uint16).astype(jnp.uint32)
  o_ref[...] = a16 | (b16 << 16)


_fuse = pl.pallas_call(
    _fuse_body,
    grid=((V + CB - 1) // CB,),
    in_specs=[pl.BlockSpec((D, CB), lambda i: (0, i))] * 2,
    out_specs=pl.BlockSpec((CB, D), lambda i: (i, 0)),
    out_shape=jax.ShapeDtypeStruct((V, D), jnp.uint32),
)


def _pool_body(usr_hbm, desc_hbm, tu, ti, ou, oi,
               idx_v, b0, b1, b2, b3, b4, b5, b6, b7,
               out_v, s0, s1, s2, s3, s4, s5, s6, s7):
  wid = lax.axis_index("s") * NC + lax.axis_index("c")
  base = wid * RPW

  bufs = (b0, b1, b2, b3, b4, b5, b6, b7)
  sems = (s0, s1, s2, s3, s4, s5, s6, s7)
  hi_mask = jnp.full((16,), 0xFFFF0000, dtype=jnp.uint32)

  def run_pass(idx_hbm, table, out_hbm):
    pltpu.sync_copy(idx_hbm.at[pl.ds(base * LP, RPW * LP)], idx_v)

    def issue(c, j):
      pltpu.async_copy(
          table.at[idx_v.at[pl.ds(c * LP, LP)]], bufs[j], sems[j])

    def wait(j):
      pltpu.make_async_copy(
          table.at[idx_v.at[pl.ds(0, LP)]], bufs[j], sems[j]).wait()

    def acc_row(c, row, j):
      buf = bufs[j]
      zero = jnp.zeros((16,), jnp.float32)

      def rbody(r, carry):
        a = list(carry)
        for q in range(4):
          v = buf[r, pl.ds(16 * q, 16)]
          lo = plsc.bitcast(v << 16, jnp.float32)
          hi = plsc.bitcast(v & hi_mask, jnp.float32)
          a[2 * q] = a[2 * q] + lo
          a[2 * q + 1] = a[2 * q + 1] + hi
        return tuple(a)

      a = lax.fori_loop(0, L, rbody, (zero,) * 8, unroll=2)
      for q in range(4):
        out_v[row, pl.ds(32 * q, 16)] = a[2 * q] * INV_L
        out_v[row, pl.ds(32 * q + 16, 16)] = a[2 * q + 1] * INV_L

    for h in range(2):
      lo = h * HALF
      for k in range(NBUF):
        issue(lo + k, k)

      def grp_body(g, carry):
        for j in range(NBUF):
          c = lo + g * NBUF + j
          wait(j)
          acc_row(c, c - lo, j)

          @pl.when(c + NBUF < lo + HALF)
          def _():
            issue(c + NBUF, j)

        return carry

      lax.fori_loop(0, NGRP, grp_body, 0)
      pltpu.sync_copy(out_v, out_hbm.at[pl.ds(base + lo, HALF)])

  run_pass(usr_hbm, tu, ou)
  run_pass(desc_hbm, ti, oi)


_pool = functools.partial(
    pl.kernel,
    out_type=[jax.ShapeDtypeStruct((B, DP), jnp.float32)] * 2,
    mesh=plsc.VectorSubcoreMesh(
        core_axis_name="c", subcore_axis_name="s",
        num_cores=NC, num_subcores=NS),
    compiler_params=pltpu.CompilerParams(
        use_tc_tiling_on_sc=False, needs_layout_passes=False),
    scratch_types=(
        [pltpu.VMEM((RPW * LP,), jnp.int32)]
        + [pltpu.VMEM((LP, D), jnp.uint32) for _ in range(NBUF)]
        + [pltpu.VMEM((HALF, DP), jnp.float32)]
        + [pltpu.SemaphoreType.DMA for _ in range(NBUF)]
    ),
)(_pool_body)


BB = 2048


def _mlp_body(u_ref, i_ref,
              W1u_ref, W1i_ref, b1_ref, W2_ref, b2_ref,
              Wam_ref, Wag_ref, ba_ref, out_ref):
  dn_e = (((1,), (0,)), ((), ()))
  dn_t = (((1,), (1,)), ((), ()))
  f32 = jnp.float32
  u = u_ref[...]
  i = i_ref[...]
  h = (lax.dot_general(u, W1u_ref[...], dn_e, preferred_element_type=f32)
       + lax.dot_general(i, W1i_ref[...], dn_e, preferred_element_type=f32)
       + b1_ref[...])
  h = jnp.maximum(h, 0.0)
  h = lax.dot_general(h, W2_ref[...], dn_t, preferred_element_type=f32) + b2_ref[...]
  h = jnp.maximum(h, 0.0)
  g = u * i
  out = (lax.dot_general(h, Wam_ref[...], dn_e, preferred_element_type=f32)
         + lax.dot_general(g, Wag_ref[...], dn_e, preferred_element_type=f32)
         + ba_ref[...])
  out_ref[...] = out


_mlp = pl.pallas_call(
    _mlp_body,
    grid=(B // BB,),
    in_specs=[
        pl.BlockSpec((BB, DP), lambda i: (i, 0)),
        pl.BlockSpec((BB, DP), lambda i: (i, 0)),
        pl.BlockSpec((DP, 64), lambda i: (0, 0)),
        pl.BlockSpec((DP, 64), lambda i: (0, 0)),
        pl.BlockSpec((1, 64), lambda i: (0, 0)),
        pl.BlockSpec((32, 64), lambda i: (0, 0)),
        pl.BlockSpec((1, 32), lambda i: (0, 0)),
        pl.BlockSpec((32, 1), lambda i: (0, 0)),
        pl.BlockSpec((DP, 1), lambda i: (0, 0)),
        pl.BlockSpec((1, 1), lambda i: (0, 0)),
    ],
    out_specs=pl.BlockSpec((BB, 1), lambda i: (i, 0)),
    out_shape=jax.ShapeDtypeStruct((B, 1), jnp.float32),
)


def kernel(usr_comments, descriptions, emb_user_mlp, emb_item_mlp,
           emb_user_gmf, emb_item_gmf, W1, b1, W2, b2, Wa, ba):
  pad_vals = jnp.arange(B * (LP - L), dtype=jnp.int32).reshape(B, LP - L) % V
  usr_p = jnp.concatenate([usr_comments, pad_vals], axis=1).reshape(-1)
  desc_p = jnp.concatenate([descriptions, pad_vals], axis=1).reshape(-1)
  pos_a = jnp.asarray(POS_A)
  pos_b = jnp.asarray(POS_B)
  W1u = jnp.zeros((DP, 64), jnp.float32).at[pos_a].set(W1[:, :D].T)
  W1i = jnp.zeros((DP, 64), jnp.float32).at[pos_a].set(W1[:, D:].T)
  Wag = jnp.zeros((DP, 1), jnp.float32).at[pos_b].set(Wa[:, 32:].T)
  Wam = Wa[:, :32].T
  tu = _fuse(emb_user_mlp.T, emb_user_gmf.T)
  ti = _fuse(emb_item_mlp.T, emb_item_gmf.T)
  pu, pi = _pool(usr_p, desc_p, tu, ti)
  return _mlp(pu, pi, W1u, W1i, b1.reshape(1, -1), W2, b2.reshape(1, -1),
              Wam, Wag, ba.reshape(1, 1))


# speedup vs baseline: 1.8004x; 1.8004x over previous
import functools

import jax
import jax.numpy as jnp
import numpy as np
from jax import lax
from jax.experimental import pallas as pl
from jax.experimental.pallas import tpu as pltpu
from jax.experimental.pallas import tpu_sc as plsc

B = 16384
L = 50
V = 1000000
LP = 56
D = 64
DP = 2 * D
NC, NS = 2, 16
NW = NC * NS
RPW = B // NW
INV_L = 1.0 / L

NBUF = 8
HALF = RPW // 2
NGRP = HALF // NBUF

_Q, _J = np.divmod(np.arange(D), 16)
POS_A = 32 * _Q + _J
POS_B = POS_A + 16


CB = 4096


def _pack16(a_ref, b_ref):
  a16 = lax.bitcast_convert_type(
      a_ref[...].T.astype(jnp.bfloat16), jnp.uint16).astype(jnp.uint32)
  b16 = lax.bitcast_convert_type(
      b_ref[...].T.astype(jnp.bfloat16), jnp.uint16).astype(jnp.uint32)
  return a16 | (b16 << 16)


def _fuse_body(au_ref, bu_ref, ai_ref, bi_ref, o_ref):
  o_ref[:, :D] = _pack16(au_ref, bu_ref)
  o_ref[:, D:] = _pack16(ai_ref, bi_ref)


_fuse = pl.pallas_call(
    _fuse_body,
    grid=((V + CB - 1) // CB,),
    in_specs=[pl.BlockSpec((D, CB), lambda i: (0, i))] * 4,
    out_specs=pl.BlockSpec((CB, DP), lambda i: (i, 0)),
    out_shape=jax.ShapeDtypeStruct((V, DP), jnp.uint32),
)


def _pool_body(usr_hbm, desc_hbm, table4, ou, oi,
               idx_v, b0, b1, b2, b3, b4, b5, b6, b7,
               out_v, s0, s1, s2, s3, s4, s5, s6, s7):
  wid = lax.axis_index("s") * NC + lax.axis_index("c")
  base = wid * RPW

  bufs = (b0, b1, b2, b3, b4, b5, b6, b7)
  sems = (s0, s1, s2, s3, s4, s5, s6, s7)
  hi_mask = jnp.full((16,), 0xFFFF0000, dtype=jnp.uint32)

  def run_pass(idx_hbm, col0, out_hbm):
    pltpu.sync_copy(idx_hbm.at[pl.ds(base * LP, RPW * LP)], idx_v)

    def issue(c, j):
      pltpu.async_copy(
          table4.at[idx_v.at[pl.ds(c * LP, LP)]], bufs[j], sems[j])

    def wait(j):
      pltpu.make_async_copy(
          table4.at[idx_v.at[pl.ds(0, LP)]], bufs[j], sems[j]).wait()

    def acc_row(c, row, j):
      buf = bufs[j]
      zero = jnp.zeros((16,), jnp.float32)

      def rbody(r, carry):
        a = list(carry)
        for q in range(4):
          v = buf[r, pl.ds(col0 + 16 * q, 16)]
          lo = plsc.bitcast(v << 16, jnp.float32)
          hi = plsc.bitcast(v & hi_mask, jnp.float32)
          a[2 * q] = a[2 * q] + lo
          a[2 * q + 1] = a[2 * q + 1] + hi
        return tuple(a)

      a = lax.fori_loop(0, L, rbody, (zero,) * 8, unroll=2)
      for q in range(4):
        out_v[row, pl.ds(32 * q, 16)] = a[2 * q] * INV_L
        out_v[row, pl.ds(32 * q + 16, 16)] = a[2 * q + 1] * INV_L

    for h in range(2):
      lo = h * HALF
      for k in range(NBUF):
        issue(lo + k, k)

      def grp_body(g, carry):
        for j in range(NBUF):
          c = lo + g * NBUF + j
          wait(j)
          acc_row(c, c - lo, j)

          @pl.when(c + NBUF < lo + HALF)
          def _():
            issue(c + NBUF, j)

        return carry

      lax.fori_loop(0, NGRP, grp_body, 0)
      pltpu.sync_copy(out_v, out_hbm.at[pl.ds(base + lo, HALF)])

  run_pass(usr_hbm, 0, ou)
  run_pass(desc_hbm, D, oi)


_pool = functools.partial(
    pl.kernel,
    out_type=[jax.ShapeDtypeStruct((B, DP), jnp.float32)] * 2,
    mesh=plsc.VectorSubcoreMesh(
        core_axis_name="c", subcore_axis_name="s",
        num_cores=NC, num_subcores=NS),
    compiler_params=pltpu.CompilerParams(
        use_tc_tiling_on_sc=False, needs_layout_passes=False),
    scratch_types=(
        [pltpu.VMEM((RPW * LP,), jnp.int32)]
        + [pltpu.VMEM((LP, DP), jnp.uint32) for _ in range(NBUF)]
        + [pltpu.VMEM((HALF, DP), jnp.float32)]
        + [pltpu.SemaphoreType.DMA for _ in range(NBUF)]
    ),
)(_pool_body)


BB = 2048


def _mlp_body(u_ref, i_ref,
              W1u_ref, W1i_ref, b1_ref, W2_ref, b2_ref,
              Wam_ref, Wag_ref, ba_ref, out_ref):
  dn_e = (((1,), (0,)), ((), ()))
  dn_t = (((1,), (1,)), ((), ()))
  f32 = jnp.float32
  u = u_ref[...]
  i = i_ref[...]
  h = (lax.dot_general(u, W1u_ref[...], dn_e, preferred_element_type=f32)
       + lax.dot_general(i, W1i_ref[...], dn_e, preferred_element_type=f32)
       + b1_ref[...])
  h = jnp.maximum(h, 0.0)
  h = lax.dot_general(h, W2_ref[...], dn_t, preferred_element_type=f32) + b2_ref[...]
  h = jnp.maximum(h, 0.0)
  g = u * i
  out = (lax.dot_general(h, Wam_ref[...], dn_e, preferred_element_type=f32)
         + lax.dot_general(g, Wag_ref[...], dn_e, preferred_element_type=f32)
         + ba_ref[...])
  out_ref[...] = out


_mlp = pl.pallas_call(
    _mlp_body,
    grid=(B // BB,),
    in_specs=[
        pl.BlockSpec((BB, DP), lambda i: (i, 0)),
        pl.BlockSpec((BB, DP), lambda i: (i, 0)),
        pl.BlockSpec((DP, 64), lambda i: (0, 0)),
        pl.BlockSpec((DP, 64), lambda i: (0, 0)),
        pl.BlockSpec((1, 64), lambda i: (0, 0)),
        pl.BlockSpec((32, 64), lambda i: (0, 0)),
        pl.BlockSpec((1, 32), lambda i: (0, 0)),
        pl.BlockSpec((32, 1), lambda i: (0, 0)),
        pl.BlockSpec((DP, 1), lambda i: (0, 0)),
        pl.BlockSpec((1, 1), lambda i: (0, 0)),
    ],
    out_specs=pl.BlockSpec((BB, 1), lambda i: (i, 0)),
    out_shape=jax.ShapeDtypeStruct((B, 1), jnp.float32),
)


def kernel(usr_comments, descriptions, emb_user_mlp, emb_item_mlp,
           emb_user_gmf, emb_item_gmf, W1, b1, W2, b2, Wa, ba):
  pad_vals = jnp.arange(B * (LP - L), dtype=jnp.int32).reshape(B, LP - L) % V
  usr_p = jnp.concatenate([usr_comments, pad_vals], axis=1).reshape(-1)
  desc_p = jnp.concatenate([descriptions, pad_vals], axis=1).reshape(-1)
  pos_a = jnp.asarray(POS_A)
  pos_b = jnp.asarray(POS_B)
  W1u = jnp.zeros((DP, 64), jnp.float32).at[pos_a].set(W1[:, :D].T)
  W1i = jnp.zeros((DP, 64), jnp.float32).at[pos_a].set(W1[:, D:].T)
  Wag = jnp.zeros((DP, 1), jnp.float32).at[pos_b].set(Wa[:, 32:].T)
  Wam = Wa[:, :32].T
  t4 = _fuse(emb_user_mlp.T, emb_user_gmf.T,
             emb_item_mlp.T, emb_item_gmf.T)
  pu, pi = _pool(usr_p, desc_p, t4)
  return _mlp(pu, pi, W1u, W1i, b1.reshape(1, -1), W2, b2.reshape(1, -1),
              Wam, Wag, ba.reshape(1, 1))


# speedup vs baseline: 1.8391x; 1.0215x over previous
import functools

import jax
import jax.numpy as jnp
import numpy as np
from jax import lax
from jax.experimental import pallas as pl
from jax.experimental.pallas import tpu as pltpu
from jax.experimental.pallas import tpu_sc as plsc

B = 16384
L = 50
V = 1000000
LP = 56
D = 64
DP = 2 * D
NC, NS = 2, 16
NW = NC * NS
RPW = B // NW
INV_L = 1.0 / L

NBUF = 8
HALF = RPW // 2
NGRP = HALF // NBUF

_Q, _J = np.divmod(np.arange(D), 16)
POS_A = 32 * _Q + _J
POS_B = POS_A + 16


CB = 4096


def _pack16(a_ref, b_ref):
  a16 = lax.bitcast_convert_type(
      a_ref[...].astype(jnp.bfloat16), jnp.uint16).astype(jnp.uint32)
  b16 = lax.bitcast_convert_type(
      b_ref[...].astype(jnp.bfloat16), jnp.uint16).astype(jnp.uint32)
  return a16 | (b16 << 16)


def _fuse_body(au_ref, bu_ref, ai_ref, bi_ref, o_ref):
  o_ref[:, :D] = _pack16(au_ref, bu_ref).T
  o_ref[:, D:] = _pack16(ai_ref, bi_ref).T


_fuse = pl.pallas_call(
    _fuse_body,
    grid=((V + CB - 1) // CB,),
    in_specs=[pl.BlockSpec((D, CB), lambda i: (0, i))] * 4,
    out_specs=pl.BlockSpec((CB, DP), lambda i: (i, 0)),
    out_shape=jax.ShapeDtypeStruct((V, DP), jnp.uint32),
)


def _pool_body(usr_hbm, desc_hbm, table4, ou, oi,
               idx_v, b0, b1, b2, b3, b4, b5, b6, b7,
               out_v, s0, s1, s2, s3, s4, s5, s6, s7):
  wid = lax.axis_index("s") * NC + lax.axis_index("c")
  base = wid * RPW

  bufs = (b0, b1, b2, b3, b4, b5, b6, b7)
  sems = (s0, s1, s2, s3, s4, s5, s6, s7)
  hi_mask = jnp.full((16,), 0xFFFF0000, dtype=jnp.uint32)

  def run_pass(idx_hbm, col0, out_hbm):
    pltpu.sync_copy(idx_hbm.at[pl.ds(base * LP, RPW * LP)], idx_v)

    def issue(c, j):
      pltpu.async_copy(
          table4.at[idx_v.at[pl.ds(c * LP, LP)]], bufs[j], sems[j])

    def wait(j):
      pltpu.make_async_copy(
          table4.at[idx_v.at[pl.ds(0, LP)]], bufs[j], sems[j]).wait()

    def acc_row(c, row, j):
      buf = bufs[j]
      zero = jnp.zeros((16,), jnp.float32)

      def rbody(r, carry):
        a = list(carry)
        for q in range(4):
          v = buf[r, pl.ds(col0 + 16 * q, 16)]
          lo = plsc.bitcast(v << 16, jnp.float32)
          hi = plsc.bitcast(v & hi_mask, jnp.float32)
          a[2 * q] = a[2 * q] + lo
          a[2 * q + 1] = a[2 * q + 1] + hi
        return tuple(a)

      a = lax.fori_loop(0, L, rbody, (zero,) * 8, unroll=2)
      for q in range(4):
        out_v[row, pl.ds(32 * q, 16)] = a[2 * q] * INV_L
        out_v[row, pl.ds(32 * q + 16, 16)] = a[2 * q + 1] * INV_L

    for h in range(2):
      lo = h * HALF
      for k in range(NBUF):
        issue(lo + k, k)

      def grp_body(g, carry):
        for j in range(NBUF):
          c = lo + g * NBUF + j
          wait(j)
          acc_row(c, c - lo, j)

          @pl.when(c + NBUF < lo + HALF)
          def _():
            issue(c + NBUF, j)

        return carry

      lax.fori_loop(0, NGRP, grp_body, 0)
      pltpu.sync_copy(out_v, out_hbm.at[pl.ds(base + lo, HALF)])

  run_pass(usr_hbm, 0, ou)
  run_pass(desc_hbm, D, oi)


_pool = functools.partial(
    pl.kernel,
    out_type=[jax.ShapeDtypeStruct((B, DP), jnp.float32)] * 2,
    mesh=plsc.VectorSubcoreMesh(
        core_axis_name="c", subcore_axis_name="s",
        num_cores=NC, num_subcores=NS),
    compiler_params=pltpu.CompilerParams(
        use_tc_tiling_on_sc=False, needs_layout_passes=False),
    scratch_types=(
        [pltpu.VMEM((RPW * LP,), jnp.int32)]
        + [pltpu.VMEM((LP, DP), jnp.uint32) for _ in range(NBUF)]
        + [pltpu.VMEM((HALF, DP), jnp.float32)]
        + [pltpu.SemaphoreType.DMA for _ in range(NBUF)]
    ),
)(_pool_body)


BB = 2048


def _mlp_body(u_ref, i_ref,
              W1u_ref, W1i_ref, b1_ref, W2_ref, b2_ref,
              Wam_ref, Wag_ref, ba_ref, out_ref):
  dn_e = (((1,), (0,)), ((), ()))
  dn_t = (((1,), (1,)), ((), ()))
  f32 = jnp.float32
  u = u_ref[...]
  i = i_ref[...]
  h = (lax.dot_general(u, W1u_ref[...], dn_e, preferred_element_type=f32)
       + lax.dot_general(i, W1i_ref[...], dn_e, preferred_element_type=f32)
       + b1_ref[...])
  h = jnp.maximum(h, 0.0)
  h = lax.dot_general(h, W2_ref[...], dn_t, preferred_element_type=f32) + b2_ref[...]
  h = jnp.maximum(h, 0.0)
  g = u * i
  out = (lax.dot_general(h, Wam_ref[...], dn_e, preferred_element_type=f32)
         + lax.dot_general(g, Wag_ref[...], dn_e, preferred_element_type=f32)
         + ba_ref[...])
  out_ref[...] = out


_mlp = pl.pallas_call(
    _mlp_body,
    grid=(B // BB,),
    in_specs=[
        pl.BlockSpec((BB, DP), lambda i: (i, 0)),
        pl.BlockSpec((BB, DP), lambda i: (i, 0)),
        pl.BlockSpec((DP, 64), lambda i: (0, 0)),
        pl.BlockSpec((DP, 64), lambda i: (0, 0)),
        pl.BlockSpec((1, 64), lambda i: (0, 0)),
        pl.BlockSpec((32, 64), lambda i: (0, 0)),
        pl.BlockSpec((1, 32), lambda i: (0, 0)),
        pl.BlockSpec((32, 1), lambda i: (0, 0)),
        pl.BlockSpec((DP, 1), lambda i: (0, 0)),
        pl.BlockSpec((1, 1), lambda i: (0, 0)),
    ],
    out_specs=pl.BlockSpec((BB, 1), lambda i: (i, 0)),
    out_shape=jax.ShapeDtypeStruct((B, 1), jnp.float32),
)


def kernel(usr_comments, descriptions, emb_user_mlp, emb_item_mlp,
           emb_user_gmf, emb_item_gmf, W1, b1, W2, b2, Wa, ba):
  pad_vals = jnp.arange(B * (LP - L), dtype=jnp.int32).reshape(B, LP - L) % V
  usr_p = jnp.concatenate([usr_comments, pad_vals], axis=1).reshape(-1)
  desc_p = jnp.concatenate([descriptions, pad_vals], axis=1).reshape(-1)
  pos_a = jnp.asarray(POS_A)
  pos_b = jnp.asarray(POS_B)
  W1u = jnp.zeros((DP, 64), jnp.float32).at[pos_a].set(W1[:, :D].T)
  W1i = jnp.zeros((DP, 64), jnp.float32).at[pos_a].set(W1[:, D:].T)
  Wag = jnp.zeros((DP, 1), jnp.float32).at[pos_b].set(Wa[:, 32:].T)
  Wam = Wa[:, :32].T
  t4 = _fuse(emb_user_mlp.T, emb_user_gmf.T,
             emb_item_mlp.T, emb_item_gmf.T)
  pu, pi = _pool(usr_p, desc_p, t4)
  return _mlp(pu, pi, W1u, W1i, b1.reshape(1, -1), W2, b2.reshape(1, -1),
              Wam, Wag, ba.reshape(1, 1))
